# baseline (speedup 1.0000x reference)
import functools

import jax
import jax.numpy as jnp
from jax import lax
from jax.experimental import pallas as pl
from jax.experimental.pallas import tpu as pltpu
from jax.experimental.pallas import tpu_sc as plsc

VOCAB = 100000
EMB = 16
B = 16384
L = 50

NUM_CORES = 2
NUM_SUBCORES = 16
NW = NUM_CORES * NUM_SUBCORES
SAMPLES_PER_W = B // NW
GROUPS_PER_W = SAMPLES_PER_W // 16

_RW = 256
_RROWS = VOCAB * EMB // _RW
_TWROW = 6400
_TW_PAD = EMB * _TWROW

_NCHAIN = 2


def _tw_body(table_ref, w_ref, b_ref, x_ref, *out_refs):
    xt_ref = out_refs[EMB]
    xt = jnp.transpose(x_ref[...])
    for j in range(L):
        xt_ref[pl.ds(j * B, B)] = xt[j, :]
    w16 = jnp.broadcast_to(w_ref[...], (EMB, EMB))
    w_tile = jnp.concatenate([w16] * EMB, axis=0)
    r_div = lax.broadcasted_iota(jnp.int32, (_RW, EMB), 0) // EMB
    c_idx = lax.broadcasted_iota(jnp.int32, (_RW, EMB), 1)
    wbig = jnp.where(r_div == c_idx, w_tile, jnp.float32(0.0))
    y = lax.dot_general(
        wbig, table_ref[...], (((0,), (1,)), ((), ())),
        preferred_element_type=jnp.float32,
    )
    y = y + b_ref[0]
    for j in range(EMB):
        out_refs[j][pl.ds(0, _RROWS)] = y[j, :]


def _compute_tw(table_r, W, b, x):
    return pl.pallas_call(
        _tw_body,
        grid=(1,),
        in_specs=[
            pl.BlockSpec((_RROWS, _RW), lambda i: (0, 0)),
            pl.BlockSpec((EMB, 1), lambda i: (0, 0)),
            pl.BlockSpec(memory_space=pltpu.SMEM),
            pl.BlockSpec((B, L), lambda i: (0, 0)),
        ],
        out_specs=(
            [pl.BlockSpec((_TWROW,), lambda i: (0,)) for _ in range(EMB)]
            + [pl.BlockSpec((B * L,), lambda i: (0,))]
        ),
        out_shape=(
            [jax.ShapeDtypeStruct((_TWROW,), jnp.float32) for _ in range(EMB)]
            + [jax.ShapeDtypeStruct((B * L,), jnp.int32)]
        ),
    )(table_r, W, b, x)


def _sc_body(*refs):
    tw_hbm = refs[:EMB]
    x_hbm, out_hbm, tw_v, x_v, out_v, sem = refs[EMB : EMB + 6]

    wid = lax.axis_index("s") * NUM_CORES + lax.axis_index("c")
    base_s = wid * SAMPLES_PER_W

    copies = [
        pltpu.async_copy(tw_hbm[j], tw_v.at[pl.ds(j * _TWROW, _TWROW)], sem)
        for j in range(EMB)
    ]
    copies.extend(
        pltpu.async_copy(
            x_hbm.at[pl.ds(j * B + base_s, SAMPLES_PER_W)],
            x_v.at[pl.ds(j * SAMPLES_PER_W, SAMPLES_PER_W)],
            sem,
        )
        for j in range(L)
    )
    for c in copies:
        c.wait()

    iota = lax.iota(jnp.int32, 16)
    inv_l = jnp.float32(1.0 / L)
    two_p23 = jnp.float32(16777216.0)

    @plsc.parallel_loop(0, GROUPS_PER_W, 1, unroll=2)
    def group(g):
        accs = [jnp.zeros((16,), jnp.float32) for _ in range(_NCHAIN)]
        for m in range(L // _NCHAIN):
            for c in range(_NCHAIN):
                j = m * _NCHAIN + c
                xi = x_v[pl.ds(j * SAMPLES_PER_W + g * 16, 16)]
                ti = (xi & 15) * _TWROW + (xi >> 4)
                accs[c] = accs[c] + plsc.load_gather(tw_v, [ti])
        while len(accs) > 1:
            accs = [a + b for a, b in zip(accs[0::2], accs[1::2])] + (
                [accs[-1]] if len(accs) % 2 else []
            )
        z = accs[0] * inv_l
        y = 1.0 / (1.0 + jnp.exp(-z))
        t = y * jnp.float32(10000.0)
        r = (t + two_p23) - two_p23
        plsc.store_scatter(out_v, [g * 16 + iota], r * jnp.float32(1e-4))

    pltpu.sync_copy(out_v, out_hbm.at[pl.ds(base_s, SAMPLES_PER_W)])


def _sc_gather(tw_stripes, x):
    mesh = plsc.VectorSubcoreMesh(core_axis_name="c", subcore_axis_name="s")
    k = functools.partial(
        pl.kernel,
        mesh=mesh,
        out_type=jax.ShapeDtypeStruct((B,), jnp.float32),
        scratch_types=[
            pltpu.VMEM((_TW_PAD,), jnp.float32),
            pltpu.VMEM((SAMPLES_PER_W * L,), jnp.int32),
            pltpu.VMEM((SAMPLES_PER_W,), jnp.float32),
            pltpu.SemaphoreType.DMA,
        ],
        compiler_params=pltpu.CompilerParams(needs_layout_passes=False),
    )(_sc_body)
    return k(*tw_stripes, x)


def kernel(x, table, W, b):
    x = x.astype(jnp.int32)
    table_r = table.reshape(_RROWS, _RW)
    *tw_stripes, x_t = _compute_tw(table_r, W, b, x)
    out = _sc_gather(tw_stripes, x_t)
    return out.reshape(B, 1)


# speedup vs baseline: 1.0107x; 1.0107x over previous
import functools

import jax
import jax.numpy as jnp
from jax import lax
from jax.experimental import pallas as pl
from jax.experimental.pallas import tpu as pltpu
from jax.experimental.pallas import tpu_sc as plsc

VOCAB = 100000
EMB = 16
B = 16384
L = 50

NUM_CORES = 2
NUM_SUBCORES = 16
NW = NUM_CORES * NUM_SUBCORES
SAMPLES_PER_W = B // NW
GROUPS_PER_W = SAMPLES_PER_W // 16

_RW = 256
_RROWS = VOCAB * EMB // _RW
_TWROW = 6400
_TW_PAD = EMB * _TWROW

_NCHAIN = 10


def _tw_body(table_ref, w_ref, b_ref, x_ref, *out_refs):
    xt_ref = out_refs[EMB]
    xt = jnp.transpose(x_ref[...])
    for j in range(L):
        xt_ref[pl.ds(j * B, B)] = xt[j, :]
    w16 = jnp.broadcast_to(w_ref[...], (EMB, EMB))
    w_tile = jnp.concatenate([w16] * EMB, axis=0)
    r_div = lax.broadcasted_iota(jnp.int32, (_RW, EMB), 0) // EMB
    c_idx = lax.broadcasted_iota(jnp.int32, (_RW, EMB), 1)
    wbig = jnp.where(r_div == c_idx, w_tile, jnp.float32(0.0))
    y = lax.dot_general(
        wbig, table_ref[...], (((0,), (1,)), ((), ())),
        preferred_element_type=jnp.float32,
    )
    y = y + b_ref[0]
    for j in range(EMB):
        out_refs[j][pl.ds(0, _RROWS)] = y[j, :]


def _compute_tw(table_r, W, b, x):
    return pl.pallas_call(
        _tw_body,
        grid=(1,),
        in_specs=[
            pl.BlockSpec((_RROWS, _RW), lambda i: (0, 0)),
            pl.BlockSpec((EMB, 1), lambda i: (0, 0)),
            pl.BlockSpec(memory_space=pltpu.SMEM),
            pl.BlockSpec((B, L), lambda i: (0, 0)),
        ],
        out_specs=(
            [pl.BlockSpec((_TWROW,), lambda i: (0,)) for _ in range(EMB)]
            + [pl.BlockSpec((B * L,), lambda i: (0,))]
        ),
        out_shape=(
            [jax.ShapeDtypeStruct((_TWROW,), jnp.float32) for _ in range(EMB)]
            + [jax.ShapeDtypeStruct((B * L,), jnp.int32)]
        ),
    )(table_r, W, b, x)


def _sc_body(*refs):
    tw_hbm = refs[:EMB]
    x_hbm, out_hbm, tw_v, x_v, out_v, sem = refs[EMB : EMB + 6]

    wid = lax.axis_index("s") * NUM_CORES + lax.axis_index("c")
    base_s = wid * SAMPLES_PER_W

    copies = [
        pltpu.async_copy(tw_hbm[j], tw_v.at[pl.ds(j * _TWROW, _TWROW)], sem)
        for j in range(EMB)
    ]
    copies.extend(
        pltpu.async_copy(
            x_hbm.at[pl.ds(j * B + base_s, SAMPLES_PER_W)],
            x_v.at[pl.ds(j * SAMPLES_PER_W, SAMPLES_PER_W)],
            sem,
        )
        for j in range(L)
    )
    for c in copies:
        c.wait()

    iota = lax.iota(jnp.int32, 16)
    inv_l = jnp.float32(1.0 / L)
    two_p23 = jnp.float32(16777216.0)

    @plsc.parallel_loop(0, GROUPS_PER_W, 1, unroll=1)
    def group(g):
        accs = [jnp.zeros((16,), jnp.float32) for _ in range(_NCHAIN)]
        for m in range(L // _NCHAIN):
            for c in range(_NCHAIN):
                j = m * _NCHAIN + c
                xi = x_v[pl.ds(j * SAMPLES_PER_W + g * 16, 16)]
                ti = (xi & 15) * _TWROW + (xi >> 4)
                accs[c] = accs[c] + plsc.load_gather(tw_v, [ti])
        while len(accs) > 1:
            accs = [a + b for a, b in zip(accs[0::2], accs[1::2])] + (
                [accs[-1]] if len(accs) % 2 else []
            )
        z = accs[0] * inv_l
        y = 1.0 / (1.0 + jnp.exp(-z))
        t = y * jnp.float32(10000.0)
        r = (t + two_p23) - two_p23
        plsc.store_scatter(out_v, [g * 16 + iota], r * jnp.float32(1e-4))

    pltpu.sync_copy(out_v, out_hbm.at[pl.ds(base_s, SAMPLES_PER_W)])


def _sc_gather(tw_stripes, x):
    mesh = plsc.VectorSubcoreMesh(core_axis_name="c", subcore_axis_name="s")
    k = functools.partial(
        pl.kernel,
        mesh=mesh,
        out_type=jax.ShapeDtypeStruct((B,), jnp.float32),
        scratch_types=[
            pltpu.VMEM((_TW_PAD,), jnp.float32),
            pltpu.VMEM((SAMPLES_PER_W * L,), jnp.int32),
            pltpu.VMEM((SAMPLES_PER_W,), jnp.float32),
            pltpu.SemaphoreType.DMA,
        ],
        compiler_params=pltpu.CompilerParams(needs_layout_passes=False),
    )(_sc_body)
    return k(*tw_stripes, x)


def kernel(x, table, W, b):
    x = x.astype(jnp.int32)
    table_r = table.reshape(_RROWS, _RW)
    *tw_stripes, x_t = _compute_tw(table_r, W, b, x)
    out = _sc_gather(tw_stripes, x_t)
    return out.reshape(B, 1)


# speedup vs baseline: 1.0261x; 1.0153x over previous
import functools

import jax
import jax.numpy as jnp
from jax import lax
from jax.experimental import pallas as pl
from jax.experimental.pallas import tpu as pltpu
from jax.experimental.pallas import tpu_sc as plsc

VOCAB = 100000
EMB = 16
B = 16384
L = 50

NUM_CORES = 2
NUM_SUBCORES = 16
NW = NUM_CORES * NUM_SUBCORES
SAMPLES_PER_W = B // NW
GROUPS_PER_W = SAMPLES_PER_W // 16

_RW = 256
_RROWS = VOCAB * EMB // _RW
_TWROW = 6400
_TW_PAD = EMB * _TWROW

_NCHAIN = 5


def _tw_body(table_ref, w_ref, b_ref, x_ref, *out_refs):
    xt_ref = out_refs[EMB]
    xt = jnp.transpose(x_ref[...])
    for w in range(NW):
        for j in range(L):
            xt_ref[pl.ds(w * (SAMPLES_PER_W * L) + j * SAMPLES_PER_W, SAMPLES_PER_W)] = (
                xt[j, w * SAMPLES_PER_W : (w + 1) * SAMPLES_PER_W]
            )
    w16 = jnp.broadcast_to(w_ref[...], (EMB, EMB))
    w_tile = jnp.concatenate([w16] * EMB, axis=0)
    r_div = lax.broadcasted_iota(jnp.int32, (_RW, EMB), 0) // EMB
    c_idx = lax.broadcasted_iota(jnp.int32, (_RW, EMB), 1)
    wbig = jnp.where(r_div == c_idx, w_tile, jnp.float32(0.0))
    y = lax.dot_general(
        wbig, table_ref[...], (((0,), (1,)), ((), ())),
        preferred_element_type=jnp.float32,
    )
    y = y + b_ref[0]
    for j in range(EMB):
        out_refs[j][pl.ds(0, _RROWS)] = y[j, :]


def _compute_tw(table_r, W, b, x):
    return pl.pallas_call(
        _tw_body,
        grid=(1,),
        in_specs=[
            pl.BlockSpec((_RROWS, _RW), lambda i: (0, 0)),
            pl.BlockSpec((EMB, 1), lambda i: (0, 0)),
            pl.BlockSpec(memory_space=pltpu.SMEM),
            pl.BlockSpec((B, L), lambda i: (0, 0)),
        ],
        out_specs=(
            [pl.BlockSpec((_TWROW,), lambda i: (0,)) for _ in range(EMB)]
            + [pl.BlockSpec((B * L,), lambda i: (0,))]
        ),
        out_shape=(
            [jax.ShapeDtypeStruct((_TWROW,), jnp.float32) for _ in range(EMB)]
            + [jax.ShapeDtypeStruct((B * L,), jnp.int32)]
        ),
    )(table_r, W, b, x)


def _sc_body(*refs):
    tw_hbm = refs[:EMB]
    x_hbm, out_hbm, tw_v, x_v, out_v, sem = refs[EMB : EMB + 6]

    wid = lax.axis_index("s") * NUM_CORES + lax.axis_index("c")
    base_s = wid * SAMPLES_PER_W

    copies = [
        pltpu.async_copy(tw_hbm[j], tw_v.at[pl.ds(j * _TWROW, _TWROW)], sem)
        for j in range(EMB)
    ]
    copies.append(
        pltpu.async_copy(
            x_hbm.at[pl.ds(base_s * L, SAMPLES_PER_W * L)], x_v, sem
        )
    )
    for c in copies:
        c.wait()

    iota = lax.iota(jnp.int32, 16)
    inv_l = jnp.float32(1.0 / L)
    two_p23 = jnp.float32(16777216.0)

    @plsc.parallel_loop(0, GROUPS_PER_W, 1, unroll=1)
    def group(g):
        accs = [jnp.zeros((16,), jnp.float32) for _ in range(_NCHAIN)]
        for m in range(L // _NCHAIN):
            for c in range(_NCHAIN):
                j = m * _NCHAIN + c
                xi = x_v[pl.ds(j * SAMPLES_PER_W + g * 16, 16)]
                ti = (xi & 15) * _TWROW + (xi >> 4)
                accs[c] = accs[c] + plsc.load_gather(tw_v, [ti])
        while len(accs) > 1:
            accs = [a + b for a, b in zip(accs[0::2], accs[1::2])] + (
                [accs[-1]] if len(accs) % 2 else []
            )
        z = accs[0] * inv_l
        y = 1.0 / (1.0 + jnp.exp(-z))
        t = y * jnp.float32(10000.0)
        r = (t + two_p23) - two_p23
        plsc.store_scatter(out_v, [g * 16 + iota], r * jnp.float32(1e-4))

    pltpu.sync_copy(out_v, out_hbm.at[pl.ds(base_s, SAMPLES_PER_W)])


def _sc_gather(tw_stripes, x):
    mesh = plsc.VectorSubcoreMesh(core_axis_name="c", subcore_axis_name="s")
    k = functools.partial(
        pl.kernel,
        mesh=mesh,
        out_type=jax.ShapeDtypeStruct((B,), jnp.float32),
        scratch_types=[
            pltpu.VMEM((_TW_PAD,), jnp.float32),
            pltpu.VMEM((SAMPLES_PER_W * L,), jnp.int32),
            pltpu.VMEM((SAMPLES_PER_W,), jnp.float32),
            pltpu.SemaphoreType.DMA,
        ],
        compiler_params=pltpu.CompilerParams(needs_layout_passes=False),
    )(_sc_body)
    return k(*tw_stripes, x)


def kernel(x, table, W, b):
    x = x.astype(jnp.int32)
    table_r = table.reshape(_RROWS, _RW)
    *tw_stripes, x_t = _compute_tw(table_r, W, b, x)
    out = _sc_gather(tw_stripes, x_t)
    return out.reshape(B, 1)


# speedup vs baseline: 1.0474x; 1.0208x over previous
import functools

import jax
import jax.numpy as jnp
from jax import lax
from jax.experimental import pallas as pl
from jax.experimental.pallas import tpu as pltpu
from jax.experimental.pallas import tpu_sc as plsc

VOCAB = 100000
EMB = 16
B = 16384
L = 50

NUM_CORES = 2
NUM_SUBCORES = 16
NW = NUM_CORES * NUM_SUBCORES
SAMPLES_PER_W = B // NW
GROUPS_PER_W = SAMPLES_PER_W // 16

_RW = 256
_RROWS = VOCAB * EMB // _RW
_TWROW = 6400
_TW_PAD = EMB * _TWROW

_NCHAIN = 5


def _tw_body(table_ref, w_ref, b_ref, x_ref, tw_ref, xt_ref):
    xt = jnp.transpose(x_ref[...])
    for w in range(NW):
        for j in range(L):
            xt_ref[pl.ds(w * (SAMPLES_PER_W * L) + j * SAMPLES_PER_W, SAMPLES_PER_W)] = (
                xt[j, w * SAMPLES_PER_W : (w + 1) * SAMPLES_PER_W]
            )
    w16 = jnp.broadcast_to(w_ref[...], (EMB, EMB))
    w_tile = jnp.concatenate([w16] * EMB, axis=0)
    r_div = lax.broadcasted_iota(jnp.int32, (_RW, EMB), 0) // EMB
    c_idx = lax.broadcasted_iota(jnp.int32, (_RW, EMB), 1)
    wbig = jnp.where(r_div == c_idx, w_tile, jnp.float32(0.0))
    y = lax.dot_general(
        wbig, table_ref[...], (((0,), (1,)), ((), ())),
        preferred_element_type=jnp.float32,
    )
    y = y + b_ref[0]
    for j in range(EMB):
        tw_ref[pl.ds(j * _TWROW, _RROWS)] = y[j, :]


def _compute_tw(table_r, W, b, x):
    return pl.pallas_call(
        _tw_body,
        grid=(1,),
        in_specs=[
            pl.BlockSpec((_RROWS, _RW), lambda i: (0, 0)),
            pl.BlockSpec((EMB, 1), lambda i: (0, 0)),
            pl.BlockSpec(memory_space=pltpu.SMEM),
            pl.BlockSpec((B, L), lambda i: (0, 0)),
        ],
        out_specs=[
            pl.BlockSpec((_TW_PAD,), lambda i: (0,)),
            pl.BlockSpec((B * L,), lambda i: (0,)),
        ],
        out_shape=[
            jax.ShapeDtypeStruct((_TW_PAD,), jnp.float32),
            jax.ShapeDtypeStruct((B * L,), jnp.int32),
        ],
    )(table_r, W, b, x)


def _sc_body(tw_hbm, x_hbm, out_hbm, tw_v, x_v, out_v, sem):
    wid = lax.axis_index("s") * NUM_CORES + lax.axis_index("c")
    base_s = wid * SAMPLES_PER_W

    copies = [pltpu.async_copy(tw_hbm, tw_v, sem)]
    copies.append(
        pltpu.async_copy(
            x_hbm.at[pl.ds(base_s * L, SAMPLES_PER_W * L)], x_v, sem
        )
    )
    for c in copies:
        c.wait()

    iota = lax.iota(jnp.int32, 16)
    inv_l = jnp.float32(1.0 / L)
    two_p23 = jnp.float32(16777216.0)

    @plsc.parallel_loop(0, GROUPS_PER_W, 1, unroll=1)
    def group(g):
        accs = [jnp.zeros((16,), jnp.float32) for _ in range(_NCHAIN)]
        for m in range(L // _NCHAIN):
            for c in range(_NCHAIN):
                j = m * _NCHAIN + c
                xi = x_v[pl.ds(j * SAMPLES_PER_W + g * 16, 16)]
                ti = (xi & 15) * _TWROW + (xi >> 4)
                accs[c] = accs[c] + plsc.load_gather(tw_v, [ti])
        while len(accs) > 1:
            accs = [a + b for a, b in zip(accs[0::2], accs[1::2])] + (
                [accs[-1]] if len(accs) % 2 else []
            )
        z = accs[0] * inv_l
        y = 1.0 / (1.0 + jnp.exp(-z))
        t = y * jnp.float32(10000.0)
        r = (t + two_p23) - two_p23
        plsc.store_scatter(out_v, [g * 16 + iota], r * jnp.float32(1e-4))

    pltpu.sync_copy(out_v, out_hbm.at[pl.ds(base_s, SAMPLES_PER_W)])


def _sc_gather(tw_stripes, x):
    mesh = plsc.VectorSubcoreMesh(core_axis_name="c", subcore_axis_name="s")
    k = functools.partial(
        pl.kernel,
        mesh=mesh,
        out_type=jax.ShapeDtypeStruct((B,), jnp.float32),
        scratch_types=[
            pltpu.VMEM((_TW_PAD,), jnp.float32),
            pltpu.VMEM((SAMPLES_PER_W * L,), jnp.int32),
            pltpu.VMEM((SAMPLES_PER_W,), jnp.float32),
            pltpu.SemaphoreType.DMA,
        ],
        compiler_params=pltpu.CompilerParams(needs_layout_passes=False),
    )(_sc_body)
    return k(tw_stripes, x)


def kernel(x, table, W, b):
    x = x.astype(jnp.int32)
    table_r = table.reshape(_RROWS, _RW)
    tw, x_t = _compute_tw(table_r, W, b, x)
    out = _sc_gather(tw, x_t)
    return out.reshape(B, 1)
